# baseline (speedup 1.0000x reference)
import dataclasses
import functools

import jax
import jax.numpy as jnp
from jax import lax
from jax.experimental import pallas as pl
from jax.experimental.pallas import tpu as pltpu
from jax.experimental.pallas import tpu_sc as plsc

N = 100000
H = 128
HP = H // 2
DE = 16
B = 2000
GRID = N // B
CH = 128
NFULL = N // CH
TAIL = N - NFULL * CH
NW = 32

BF = jnp.bfloat16
F32 = jnp.float32


def _pack(x):
    xb = x.astype(BF)
    lo = lax.bitcast_convert_type(xb[:, :HP], jnp.uint16).astype(jnp.uint32)
    hi = lax.bitcast_convert_type(xb[:, HP:], jnp.uint16).astype(jnp.uint32)
    return lax.bitcast_convert_type(lo | (hi << 16), jnp.int32)


def _unpack(w):
    wu = lax.bitcast_convert_type(w, jnp.uint32)
    lo = lax.bitcast_convert_type((wu & 0xFFFF).astype(jnp.uint16), BF)
    hi = lax.bitcast_convert_type((wu >> 16).astype(jnp.uint16), BF)
    return jnp.concatenate([lo, hi], axis=1)


def _tc_encode(nf, ef, W_node, bn, Wmpa0, Wmpc0, Wmpb, W_edge, be, bmp):
    def body(nf_ref, ef_ref, wn_ref, bn_ref, wa_ref, wc_ref, wb_ref, we_ref,
             be_ref, bmp_ref, n_ref, a_ref, c_ref, e0_ref, e1_ref, e2_ref):
        n0 = jnp.dot(nf_ref[...].astype(BF), wn_ref[...].astype(BF),
                     preferred_element_type=F32) + bn_ref[...]
        n0b = n0.astype(BF)
        n_ref[...] = _pack(n0)
        a_ref[...] = _pack(jnp.dot(n0b, wa_ref[...].astype(BF),
                                   preferred_element_type=F32))
        c_ref[...] = _pack(jnp.dot(n0b, wc_ref[...].astype(BF),
                                   preferred_element_type=F32))
        ef = ef_ref[...].astype(BF)
        for l, er in enumerate((e0_ref, e1_ref, e2_ref)):
            wb = wb_ref[l]
            wfold = jnp.dot(we_ref[...], wb, preferred_element_type=F32)
            bfold = jnp.dot(be_ref[...], wb,
                            preferred_element_type=F32) + bmp_ref[l]
            er[...] = _pack(jnp.dot(ef, wfold.astype(BF),
                                    preferred_element_type=F32) + bfold)

    row = lambda i: (i, 0)
    fixed = lambda i: (0, 0)
    fixed3 = lambda i: (0, 0, 0)
    out = jax.ShapeDtypeStruct((N, HP), jnp.int32)
    return pl.pallas_call(
        body,
        grid=(GRID,),
        in_specs=[
            pl.BlockSpec((B, H), row),
            pl.BlockSpec((B, DE), row),
            pl.BlockSpec((H, H), fixed),
            pl.BlockSpec((1, H), fixed),
            pl.BlockSpec((H, H), fixed),
            pl.BlockSpec((H, H), fixed),
            pl.BlockSpec((3, H, H), fixed3),
            pl.BlockSpec((DE, H), fixed),
            pl.BlockSpec((1, H), fixed),
            pl.BlockSpec((3, 1, H), fixed3),
        ],
        out_specs=[pl.BlockSpec((B, HP), row)] * 6,
        out_shape=[out] * 6,
        compiler_params=pltpu.CompilerParams(
            dimension_semantics=("parallel",)),
    )(nf, ef, W_node, bn, Wmpa0, Wmpc0, Wmpb, W_edge, be, bmp)


def _tc_proj(nodes, Wa, Wc):
    def body(n_ref, wa_ref, wc_ref, a_ref, c_ref):
        n = _unpack(n_ref[...])
        a_ref[...] = _pack(jnp.dot(n, wa_ref[...].astype(BF),
                                   preferred_element_type=F32))
        c_ref[...] = _pack(jnp.dot(n, wc_ref[...].astype(BF),
                                   preferred_element_type=F32))

    row = lambda i: (i, 0)
    fixed = lambda i: (0, 0)
    out = jax.ShapeDtypeStruct((N, HP), jnp.int32)
    return pl.pallas_call(
        body,
        grid=(GRID,),
        in_specs=[
            pl.BlockSpec((B, HP), row),
            pl.BlockSpec((H, H), fixed),
            pl.BlockSpec((H, H), fixed),
        ],
        out_specs=[pl.BlockSpec((B, HP), row)] * 2,
        out_shape=[out] * 2,
        compiler_params=pltpu.CompilerParams(
            dimension_semantics=("parallel",)),
    )(nodes, Wa, Wc)


def _sc_layer(A, C, E, nodes, src, dst):
    mesh = plsc.VectorSubcoreMesh(core_axis_name="c", subcore_axis_name="s")
    cp = pltpu.CompilerParams(use_tc_tiling_on_sc=False)
    if "needs_layout_passes" in pltpu.CompilerParams.__dataclass_fields__:
        cp = dataclasses.replace(cp, needs_layout_passes=False)

    @functools.partial(
        pl.kernel,
        out_type=jax.ShapeDtypeStruct((N, HP), jnp.int32),
        mesh=mesh,
        compiler_params=cp,
        scratch_types=[
            pltpu.VMEM((CH,), jnp.int32),
            pltpu.VMEM((CH,), jnp.int32),
            pltpu.VMEM((CH, HP), jnp.int32),
            pltpu.VMEM((CH, HP), jnp.int32),
            pltpu.VMEM((CH, HP), jnp.int32),
            pltpu.VMEM((CH, HP), jnp.int32),
            pltpu.SemaphoreType.DMA,
            pltpu.SemaphoreType.DMA,
            pltpu.SemaphoreType.DMA,
        ],
    )
    def k(a_hbm, c_hbm, e_hbm, n_hbm, src_hbm, dst_hbm, out_hbm,
          si, di, ga, gc, ev, nv, sem_i, sem_a, sem_c):
        wid = lax.axis_index("s") * 2 + lax.axis_index("c")

        def do_chunk(base, ch):
            rows = pl.ds(0, ch)
            cp_s = pltpu.async_copy(src_hbm.at[pl.ds(base, ch)],
                                    si.at[rows], sem_i)
            cp_d = pltpu.async_copy(dst_hbm.at[pl.ds(base, ch)],
                                    di.at[rows], sem_i)
            cp_s.wait()
            cp_d.wait()
            cp_a = pltpu.async_copy(a_hbm.at[si.at[rows]], ga.at[rows], sem_a)
            cp_c = pltpu.async_copy(c_hbm.at[di.at[rows]], gc.at[rows], sem_c)
            cp_e = pltpu.async_copy(e_hbm.at[pl.ds(base, ch)],
                                    ev.at[rows], sem_i)
            cp_n = pltpu.async_copy(n_hbm.at[pl.ds(base, ch)],
                                    nv.at[rows], sem_i)
            cp_a.wait()
            cp_c.wait()
            cp_e.wait()
            cp_n.wait()

            @pl.loop(0, ch)
            def _(r):
                for j in range(HP // 16):
                    sl = pl.ds(j * 16, 16)
                    a = plsc.bitcast(ga[r, sl], BF)
                    c = plsc.bitcast(gc[r, sl], BF)
                    e = plsc.bitcast(ev[r, sl], BF)
                    n = plsc.bitcast(nv[r, sl], BF)
                    m = jnp.maximum(a + c + e, BF(0))
                    ev[r, sl] = plsc.bitcast(n + m, jnp.int32)

            pltpu.sync_copy(ev.at[rows], out_hbm.at[pl.ds(base, ch)])

        @pl.loop(wid, NFULL, step=NW)
        def _(ci):
            do_chunk(ci * CH, CH)

        @pl.when(wid == NW - 1)
        def _():
            do_chunk(NFULL * CH, TAIL)

    return k(A, C, E, nodes, src, dst)


def _tc_readout(nodes, W_r1, b1, W_r2, b2, w3row, b3, W_ap, bap):
    def body(n_ref, w1_ref, b1_ref, w2_ref, b2_ref, w3_ref, b3_ref,
             wap_ref, bap_ref, risk_ref, ap_ref):
        n = _unpack(n_ref[...])
        h = jnp.maximum(jnp.dot(n, w1_ref[...].astype(BF),
                                preferred_element_type=F32)
                        + b1_ref[...], 0.0)
        h = jnp.maximum(jnp.dot(h.astype(BF), w2_ref[...].astype(BF),
                                preferred_element_type=F32)
                        + b2_ref[...], 0.0)
        r = jnp.sum(h * w3_ref[...], axis=1, keepdims=True) + b3_ref[...]
        risk_ref[...] = jax.nn.sigmoid(r)
        ap_ref[...] = jnp.dot(n, wap_ref[...].astype(BF),
                              preferred_element_type=F32) + bap_ref[...]

    row = lambda i: (i, 0)
    fixed = lambda i: (0, 0)
    return pl.pallas_call(
        body,
        grid=(GRID,),
        in_specs=[
            pl.BlockSpec((B, HP), row),
            pl.BlockSpec((H, H), fixed),
            pl.BlockSpec((1, H), fixed),
            pl.BlockSpec((H, 64), fixed),
            pl.BlockSpec((1, 64), fixed),
            pl.BlockSpec((1, 64), fixed),
            pl.BlockSpec((1, 1), fixed),
            pl.BlockSpec((H, H), fixed),
            pl.BlockSpec((1, H), fixed),
        ],
        out_specs=[pl.BlockSpec((B, 1), row), pl.BlockSpec((B, H), row)],
        out_shape=[jax.ShapeDtypeStruct((N, 1), F32),
                   jax.ShapeDtypeStruct((N, H), F32)],
        compiler_params=pltpu.CompilerParams(
            dimension_semantics=("parallel",)),
    )(nodes, W_r1, b1, W_r2, b2, w3row, b3, W_ap, bap)


def kernel(node_features, edge_features, edge_index, W_node, b_node, W_edge,
           b_edge, W_mp, b_mp, W_r1, b_r1, W_r2, b_r2, W_r3, b_r3, W_ap, b_ap):
    src = edge_index[0]
    dst = edge_index[1]
    Wmpa = W_mp[:, 0:H, :]
    Wmpb = W_mp[:, H:2 * H, :]
    Wmpc = W_mp[:, 2 * H:3 * H, :]

    nodes, A, C, E0, E1, E2 = _tc_encode(
        node_features, edge_features, W_node, b_node.reshape(1, H),
        Wmpa[0], Wmpc[0], Wmpb, W_edge, b_edge.reshape(1, H),
        b_mp.reshape(3, 1, H))

    for l, E in enumerate((E0, E1, E2)):
        nodes = _sc_layer(A, C, E, nodes, src, dst)
        if l < 2:
            A, C = _tc_proj(nodes, Wmpa[l + 1], Wmpc[l + 1])

    risk, ap = _tc_readout(
        nodes, W_r1, b_r1.reshape(1, H), W_r2, b_r2.reshape(1, 64),
        W_r3.reshape(1, 64), b_r3.reshape(1, 1), W_ap, b_ap.reshape(1, H))
    return (risk, ap)


# speedup vs baseline: 1.2353x; 1.2353x over previous
import functools

import jax
import jax.numpy as jnp
from jax import lax
from jax.experimental import pallas as pl
from jax.experimental.pallas import tpu as pltpu
from jax.experimental.pallas import tpu_sc as plsc

N = 100000
H = 128
DE = 16
B = 2000
GRID = N // B
CH = 128
NFULL = N // CH
TAIL = N - NFULL * CH
NW = 32

BF = jnp.bfloat16
F32 = jnp.float32


def _tc_encode(nf, ef, W_node, bn, Wmpa0, Wmpc0, Wmpb, W_edge, be, bmp):
    def body(nf_ref, ef_ref, wn_ref, bn_ref, wa_ref, wc_ref, wb_ref, we_ref,
             be_ref, bmp_ref, n_ref, a_ref, c_ref, e0_ref, e1_ref, e2_ref):
        n0 = jnp.dot(nf_ref[...].astype(BF), wn_ref[...].astype(BF),
                     preferred_element_type=F32) + bn_ref[...]
        n0b = n0.astype(BF)
        n_ref[...] = n0
        a_ref[...] = jnp.dot(n0b, wa_ref[...].astype(BF),
                             preferred_element_type=F32)
        c_ref[...] = jnp.dot(n0b, wc_ref[...].astype(BF),
                             preferred_element_type=F32)
        ef = ef_ref[...].astype(BF)
        for l, er in enumerate((e0_ref, e1_ref, e2_ref)):
            wb = wb_ref[l]
            wfold = jnp.dot(we_ref[...], wb, preferred_element_type=F32)
            bfold = jnp.dot(be_ref[...], wb,
                            preferred_element_type=F32) + bmp_ref[l]
            er[...] = jnp.dot(ef, wfold.astype(BF),
                              preferred_element_type=F32) + bfold

    row = lambda i: (i, 0)
    fixed = lambda i: (0, 0)
    fixed3 = lambda i: (0, 0, 0)
    out = jax.ShapeDtypeStruct((N, H), F32)
    return pl.pallas_call(
        body,
        grid=(GRID,),
        in_specs=[
            pl.BlockSpec((B, H), row),
            pl.BlockSpec((B, DE), row),
            pl.BlockSpec((H, H), fixed),
            pl.BlockSpec((1, H), fixed),
            pl.BlockSpec((H, H), fixed),
            pl.BlockSpec((H, H), fixed),
            pl.BlockSpec((3, H, H), fixed3),
            pl.BlockSpec((DE, H), fixed),
            pl.BlockSpec((1, H), fixed),
            pl.BlockSpec((3, 1, H), fixed3),
        ],
        out_specs=[pl.BlockSpec((B, H), row)] * 6,
        out_shape=[out] * 6,
        compiler_params=pltpu.CompilerParams(
            dimension_semantics=("parallel",)),
    )(nf, ef, W_node, bn, Wmpa0, Wmpc0, Wmpb, W_edge, be, bmp)


def _tc_proj(nodes, Wa, Wc):
    def body(n_ref, wa_ref, wc_ref, a_ref, c_ref):
        n = n_ref[...].astype(BF)
        a_ref[...] = jnp.dot(n, wa_ref[...].astype(BF),
                             preferred_element_type=F32)
        c_ref[...] = jnp.dot(n, wc_ref[...].astype(BF),
                             preferred_element_type=F32)

    row = lambda i: (i, 0)
    fixed = lambda i: (0, 0)
    out = jax.ShapeDtypeStruct((N, H), F32)
    return pl.pallas_call(
        body,
        grid=(GRID,),
        in_specs=[
            pl.BlockSpec((B, H), row),
            pl.BlockSpec((H, H), fixed),
            pl.BlockSpec((H, H), fixed),
        ],
        out_specs=[pl.BlockSpec((B, H), row)] * 2,
        out_shape=[out] * 2,
        compiler_params=pltpu.CompilerParams(
            dimension_semantics=("parallel",)),
    )(nodes, Wa, Wc)


def _sc_layer(A, C, E, nodes, src, dst):
    mesh = plsc.VectorSubcoreMesh(core_axis_name="c", subcore_axis_name="s")

    @functools.partial(
        pl.kernel,
        out_type=jax.ShapeDtypeStruct((N, H), F32),
        mesh=mesh,
        scratch_types=[
            pltpu.VMEM((CH,), jnp.int32),
            pltpu.VMEM((CH,), jnp.int32),
            pltpu.VMEM((CH, H), F32),
            pltpu.VMEM((CH, H), F32),
            pltpu.VMEM((CH, H), F32),
            pltpu.VMEM((CH, H), F32),
            pltpu.SemaphoreType.DMA,
            pltpu.SemaphoreType.DMA,
            pltpu.SemaphoreType.DMA,
        ],
    )
    def k(a_hbm, c_hbm, e_hbm, n_hbm, src_hbm, dst_hbm, out_hbm,
          si, di, ga, gc, ev, nv, sem_i, sem_a, sem_c):
        wid = lax.axis_index("s") * 2 + lax.axis_index("c")

        def do_chunk(base, ch):
            rows = pl.ds(0, ch)
            cp_s = pltpu.async_copy(src_hbm.at[pl.ds(base, ch)],
                                    si.at[rows], sem_i)
            cp_d = pltpu.async_copy(dst_hbm.at[pl.ds(base, ch)],
                                    di.at[rows], sem_i)
            cp_s.wait()
            cp_d.wait()
            cp_a = pltpu.async_copy(a_hbm.at[si.at[rows]], ga.at[rows], sem_a)
            cp_c = pltpu.async_copy(c_hbm.at[di.at[rows]], gc.at[rows], sem_c)
            cp_e = pltpu.async_copy(e_hbm.at[pl.ds(base, ch)],
                                    ev.at[rows], sem_i)
            cp_n = pltpu.async_copy(n_hbm.at[pl.ds(base, ch)],
                                    nv.at[rows], sem_i)
            cp_a.wait()
            cp_c.wait()
            cp_e.wait()
            cp_n.wait()

            @pl.loop(0, ch)
            def _(r):
                for j in range(H // 16):
                    sl = pl.ds(j * 16, 16)
                    m = ga[r, sl] + gc[r, sl] + ev[r, sl]
                    ev[r, sl] = nv[r, sl] + jnp.maximum(m, 0.0)

            pltpu.sync_copy(ev.at[rows], out_hbm.at[pl.ds(base, ch)])

        @pl.loop(wid, NFULL, step=NW)
        def _(ci):
            do_chunk(ci * CH, CH)

        @pl.when(wid == NW - 1)
        def _():
            do_chunk(NFULL * CH, TAIL)

    return k(A, C, E, nodes, src, dst)


def _tc_readout(nodes, W_r1, b1, W_r2, b2, w3row, b3, W_ap, bap):
    def body(n_ref, w1_ref, b1_ref, w2_ref, b2_ref, w3_ref, b3_ref,
             wap_ref, bap_ref, risk_ref, ap_ref):
        n = n_ref[...].astype(BF)
        h = jnp.maximum(jnp.dot(n, w1_ref[...].astype(BF),
                                preferred_element_type=F32)
                        + b1_ref[...], 0.0)
        h = jnp.maximum(jnp.dot(h.astype(BF), w2_ref[...].astype(BF),
                                preferred_element_type=F32)
                        + b2_ref[...], 0.0)
        r = jnp.sum(h * w3_ref[...], axis=1, keepdims=True) + b3_ref[...]
        risk_ref[...] = jax.nn.sigmoid(r)
        ap_ref[...] = jnp.dot(n, wap_ref[...].astype(BF),
                              preferred_element_type=F32) + bap_ref[...]

    row = lambda i: (i, 0)
    fixed = lambda i: (0, 0)
    return pl.pallas_call(
        body,
        grid=(GRID,),
        in_specs=[
            pl.BlockSpec((B, H), row),
            pl.BlockSpec((H, H), fixed),
            pl.BlockSpec((1, H), fixed),
            pl.BlockSpec((H, 64), fixed),
            pl.BlockSpec((1, 64), fixed),
            pl.BlockSpec((1, 64), fixed),
            pl.BlockSpec((1, 1), fixed),
            pl.BlockSpec((H, H), fixed),
            pl.BlockSpec((1, H), fixed),
        ],
        out_specs=[pl.BlockSpec((B, 1), row), pl.BlockSpec((B, H), row)],
        out_shape=[jax.ShapeDtypeStruct((N, 1), F32),
                   jax.ShapeDtypeStruct((N, H), F32)],
        compiler_params=pltpu.CompilerParams(
            dimension_semantics=("parallel",)),
    )(nodes, W_r1, b1, W_r2, b2, w3row, b3, W_ap, bap)


def kernel(node_features, edge_features, edge_index, W_node, b_node, W_edge,
           b_edge, W_mp, b_mp, W_r1, b_r1, W_r2, b_r2, W_r3, b_r3, W_ap, b_ap):
    src = edge_index[0]
    dst = edge_index[1]
    Wmpa = W_mp[:, 0:H, :]
    Wmpb = W_mp[:, H:2 * H, :]
    Wmpc = W_mp[:, 2 * H:3 * H, :]

    nodes, A, C, E0, E1, E2 = _tc_encode(
        node_features, edge_features, W_node, b_node.reshape(1, H),
        Wmpa[0], Wmpc[0], Wmpb, W_edge, b_edge.reshape(1, H),
        b_mp.reshape(3, 1, H))

    for l, E in enumerate((E0, E1, E2)):
        nodes = _sc_layer(A, C, E, nodes, src, dst)
        if l < 2:
            A, C = _tc_proj(nodes, Wmpa[l + 1], Wmpc[l + 1])

    risk, ap = _tc_readout(
        nodes, W_r1, b_r1.reshape(1, H), W_r2, b_r2.reshape(1, 64),
        W_r3.reshape(1, 64), b_r3.reshape(1, 1), W_ap, b_ap.reshape(1, H))
    return (risk, ap)


# speedup vs baseline: 1.6758x; 1.3566x over previous
import functools

import jax
import jax.numpy as jnp
from jax import lax
from jax.experimental import pallas as pl
from jax.experimental.pallas import tpu as pltpu
from jax.experimental.pallas import tpu_sc as plsc

N = 100000
H = 128
DE = 16
B = 2000
GRID = N // B
CH = 112
NFULL = N // CH
TAIL = N - NFULL * CH
NW = 32
KMAX = -(-NFULL // NW)

BF = jnp.bfloat16
F32 = jnp.float32


def _tc_encode(nf, ef, W_node, bn, Wmpa0, Wmpc0, Wmpb, W_edge, be, bmp):
    def body(nf_ref, ef_ref, wn_ref, bn_ref, wa_ref, wc_ref, wb_ref, we_ref,
             be_ref, bmp_ref, n_ref, a_ref, c_ref, e0_ref, e1_ref, e2_ref):
        n0 = jnp.dot(nf_ref[...].astype(BF), wn_ref[...].astype(BF),
                     preferred_element_type=F32) + bn_ref[...]
        n0b = n0.astype(BF)
        n_ref[...] = n0
        a_ref[...] = jnp.dot(n0b, wa_ref[...].astype(BF),
                             preferred_element_type=F32)
        c_ref[...] = jnp.dot(n0b, wc_ref[...].astype(BF),
                             preferred_element_type=F32)
        ef = ef_ref[...].astype(BF)
        for l, er in enumerate((e0_ref, e1_ref, e2_ref)):
            wb = wb_ref[l]
            wfold = jnp.dot(we_ref[...], wb, preferred_element_type=F32)
            bfold = jnp.dot(be_ref[...], wb,
                            preferred_element_type=F32) + bmp_ref[l]
            er[...] = jnp.dot(ef, wfold.astype(BF),
                              preferred_element_type=F32) + bfold

    row = lambda i: (i, 0)
    fixed = lambda i: (0, 0)
    fixed3 = lambda i: (0, 0, 0)
    out = jax.ShapeDtypeStruct((N, H), F32)
    return pl.pallas_call(
        body,
        grid=(GRID,),
        in_specs=[
            pl.BlockSpec((B, H), row),
            pl.BlockSpec((B, DE), row),
            pl.BlockSpec((H, H), fixed),
            pl.BlockSpec((1, H), fixed),
            pl.BlockSpec((H, H), fixed),
            pl.BlockSpec((H, H), fixed),
            pl.BlockSpec((3, H, H), fixed3),
            pl.BlockSpec((DE, H), fixed),
            pl.BlockSpec((1, H), fixed),
            pl.BlockSpec((3, 1, H), fixed3),
        ],
        out_specs=[pl.BlockSpec((B, H), row)] * 6,
        out_shape=[out] * 6,
        compiler_params=pltpu.CompilerParams(
            dimension_semantics=("parallel",)),
    )(nf, ef, W_node, bn, Wmpa0, Wmpc0, Wmpb, W_edge, be, bmp)


def _tc_proj(nodes, Wa, Wc):
    def body(n_ref, wa_ref, wc_ref, a_ref, c_ref):
        n = n_ref[...].astype(BF)
        a_ref[...] = jnp.dot(n, wa_ref[...].astype(BF),
                             preferred_element_type=F32)
        c_ref[...] = jnp.dot(n, wc_ref[...].astype(BF),
                             preferred_element_type=F32)

    row = lambda i: (i, 0)
    fixed = lambda i: (0, 0)
    out = jax.ShapeDtypeStruct((N, H), F32)
    return pl.pallas_call(
        body,
        grid=(GRID,),
        in_specs=[
            pl.BlockSpec((B, H), row),
            pl.BlockSpec((H, H), fixed),
            pl.BlockSpec((H, H), fixed),
        ],
        out_specs=[pl.BlockSpec((B, H), row)] * 2,
        out_shape=[out] * 2,
        compiler_params=pltpu.CompilerParams(
            dimension_semantics=("parallel",)),
    )(nodes, Wa, Wc)


def _sc_layer(A, C, E, nodes, src, dst):
    mesh = plsc.VectorSubcoreMesh(core_axis_name="c", subcore_axis_name="s")

    @functools.partial(
        pl.kernel,
        out_type=jax.ShapeDtypeStruct((N, H), F32),
        mesh=mesh,
        scratch_types=[
            pltpu.VMEM((CH,), jnp.int32), pltpu.VMEM((CH,), jnp.int32),
            pltpu.VMEM((CH,), jnp.int32), pltpu.VMEM((CH,), jnp.int32),
            pltpu.VMEM((CH, H), F32), pltpu.VMEM((CH, H), F32),
            pltpu.VMEM((CH, H), F32), pltpu.VMEM((CH, H), F32),
            pltpu.VMEM((CH, H), F32), pltpu.VMEM((CH, H), F32),
            pltpu.VMEM((CH, H), F32), pltpu.VMEM((CH, H), F32),
            pltpu.SemaphoreType.DMA, pltpu.SemaphoreType.DMA,
            pltpu.SemaphoreType.DMA, pltpu.SemaphoreType.DMA,
            pltpu.SemaphoreType.DMA, pltpu.SemaphoreType.DMA,
        ],
    )
    def k(a_hbm, c_hbm, e_hbm, n_hbm, src_hbm, dst_hbm, out_hbm,
          si0, si1, di0, di1, ga0, ga1, gc0, gc1, ev0, ev1, nv0, nv1,
          semi0, semi1, semg0, semg1, semo0, semo1):
        wid = lax.axis_index("s") * 2 + lax.axis_index("c")
        kw = (NFULL + NW - 1 - wid) // NW

        sis = (si0, si1)
        dis = (di0, di1)
        gas = (ga0, ga1)
        gcs = (gc0, gc1)
        evs = (ev0, ev1)
        nvs = (nv0, nv1)
        semis = (semi0, semi1)
        semgs = (semg0, semg1)
        semos = (semo0, semo1)

        def cbase(kc):
            return (wid + kc * NW) * CH

        def issue_idx(kc, p):
            b = cbase(kc)
            pltpu.async_copy(src_hbm.at[pl.ds(b, CH)], sis[p], semis[p])
            pltpu.async_copy(dst_hbm.at[pl.ds(b, CH)], dis[p], semis[p])

        def wait_idx(p):
            pltpu.make_async_copy(src_hbm.at[pl.ds(0, CH)], sis[p],
                                  semis[p]).wait()
            pltpu.make_async_copy(dst_hbm.at[pl.ds(0, CH)], dis[p],
                                  semis[p]).wait()

        def issue_main(kc, p):
            b = cbase(kc)
            pltpu.async_copy(a_hbm.at[sis[p]], gas[p], semgs[p])
            pltpu.async_copy(c_hbm.at[dis[p]], gcs[p], semgs[p])
            pltpu.async_copy(e_hbm.at[pl.ds(b, CH)], evs[p], semgs[p])
            pltpu.async_copy(n_hbm.at[pl.ds(b, CH)], nvs[p], semgs[p])

        def wait_main(p):
            pltpu.make_async_copy(a_hbm.at[sis[p]], gas[p], semgs[p]).wait()
            pltpu.make_async_copy(c_hbm.at[dis[p]], gcs[p], semgs[p]).wait()
            pltpu.make_async_copy(e_hbm.at[pl.ds(0, CH)], evs[p],
                                  semgs[p]).wait()
            pltpu.make_async_copy(n_hbm.at[pl.ds(0, CH)], nvs[p],
                                  semgs[p]).wait()

        def compute(p):
            ga, gc, ev, nv = gas[p], gcs[p], evs[p], nvs[p]

            @pl.loop(0, CH)
            def _(r):
                for j in range(H // 16):
                    sl = pl.ds(j * 16, 16)
                    m = ga[r, sl] + gc[r, sl] + ev[r, sl]
                    ga[r, sl] = nv[r, sl] + jnp.maximum(m, 0.0)

        def issue_out(kc, p):
            pltpu.async_copy(gas[p], out_hbm.at[pl.ds(cbase(kc), CH)],
                             semos[p])

        def wait_out(p):
            pltpu.make_async_copy(gas[p], out_hbm.at[pl.ds(0, CH)],
                                  semos[p]).wait()

        issue_idx(0, 0)
        wait_idx(0)
        issue_main(0, 0)
        issue_idx(1, 1)

        @pl.loop(0, KMAX, step=2)
        def _(t):
            for b2 in range(2):
                p = b2
                q = 1 - b2

                def body(kc=t + b2, p=p, q=q):
                    @pl.when(kc + 1 < kw)
                    def _():
                        wait_idx(q)

                        @pl.when(kc >= 1)
                        def _():
                            wait_out(q)

                        issue_main(kc + 1, q)

                    wait_main(p)

                    @pl.when(kc + 2 < kw)
                    def _():
                        issue_idx(kc + 2, p)

                    compute(p)
                    issue_out(kc, p)

                @pl.when(t + b2 < kw)
                def _():
                    body()

        wait_out(0)
        wait_out(1)

        @pl.when(wid == NW - 1)
        def _():
            base = NFULL * CH
            rows = pl.ds(0, TAIL)
            pltpu.async_copy(src_hbm.at[pl.ds(base, TAIL)],
                             si0.at[rows], semi0)
            pltpu.async_copy(dst_hbm.at[pl.ds(base, TAIL)],
                             di0.at[rows], semi0)
            pltpu.make_async_copy(src_hbm.at[pl.ds(0, TAIL)], si0.at[rows],
                                  semi0).wait()
            pltpu.make_async_copy(dst_hbm.at[pl.ds(0, TAIL)], di0.at[rows],
                                  semi0).wait()
            pltpu.async_copy(a_hbm.at[si0.at[rows]], ga0.at[rows], semg0)
            pltpu.async_copy(c_hbm.at[di0.at[rows]], gc0.at[rows], semg0)
            pltpu.async_copy(e_hbm.at[pl.ds(base, TAIL)], ev0.at[rows], semg0)
            pltpu.async_copy(n_hbm.at[pl.ds(base, TAIL)], nv0.at[rows], semg0)
            pltpu.make_async_copy(a_hbm.at[si0.at[rows]], ga0.at[rows],
                                  semg0).wait()
            pltpu.make_async_copy(c_hbm.at[di0.at[rows]], gc0.at[rows],
                                  semg0).wait()
            pltpu.make_async_copy(e_hbm.at[pl.ds(0, TAIL)], ev0.at[rows],
                                  semg0).wait()
            pltpu.make_async_copy(n_hbm.at[pl.ds(0, TAIL)], nv0.at[rows],
                                  semg0).wait()

            @pl.loop(0, TAIL)
            def _(r):
                for j in range(H // 16):
                    sl = pl.ds(j * 16, 16)
                    m = ga0[r, sl] + gc0[r, sl] + ev0[r, sl]
                    ga0[r, sl] = nv0[r, sl] + jnp.maximum(m, 0.0)

            pltpu.sync_copy(ga0.at[rows], out_hbm.at[pl.ds(base, TAIL)])

    return k(A, C, E, nodes, src, dst)


def _tc_readout(nodes, W_r1, b1, W_r2, b2, w3row, b3, W_ap, bap):
    def body(n_ref, w1_ref, b1_ref, w2_ref, b2_ref, w3_ref, b3_ref,
             wap_ref, bap_ref, risk_ref, ap_ref):
        n = n_ref[...].astype(BF)
        h = jnp.maximum(jnp.dot(n, w1_ref[...].astype(BF),
                                preferred_element_type=F32)
                        + b1_ref[...], 0.0)
        h = jnp.maximum(jnp.dot(h.astype(BF), w2_ref[...].astype(BF),
                                preferred_element_type=F32)
                        + b2_ref[...], 0.0)
        r = jnp.sum(h * w3_ref[...], axis=1, keepdims=True) + b3_ref[...]
        risk_ref[...] = jax.nn.sigmoid(r)
        ap_ref[...] = jnp.dot(n, wap_ref[...].astype(BF),
                              preferred_element_type=F32) + bap_ref[...]

    row = lambda i: (i, 0)
    fixed = lambda i: (0, 0)
    return pl.pallas_call(
        body,
        grid=(GRID,),
        in_specs=[
            pl.BlockSpec((B, H), row),
            pl.BlockSpec((H, H), fixed),
            pl.BlockSpec((1, H), fixed),
            pl.BlockSpec((H, 64), fixed),
            pl.BlockSpec((1, 64), fixed),
            pl.BlockSpec((1, 64), fixed),
            pl.BlockSpec((1, 1), fixed),
            pl.BlockSpec((H, H), fixed),
            pl.BlockSpec((1, H), fixed),
        ],
        out_specs=[pl.BlockSpec((B, 1), row), pl.BlockSpec((B, H), row)],
        out_shape=[jax.ShapeDtypeStruct((N, 1), F32),
                   jax.ShapeDtypeStruct((N, H), F32)],
        compiler_params=pltpu.CompilerParams(
            dimension_semantics=("parallel",)),
    )(nodes, W_r1, b1, W_r2, b2, w3row, b3, W_ap, bap)


def kernel(node_features, edge_features, edge_index, W_node, b_node, W_edge,
           b_edge, W_mp, b_mp, W_r1, b_r1, W_r2, b_r2, W_r3, b_r3, W_ap, b_ap):
    src = edge_index[0]
    dst = edge_index[1]
    Wmpa = W_mp[:, 0:H, :]
    Wmpb = W_mp[:, H:2 * H, :]
    Wmpc = W_mp[:, 2 * H:3 * H, :]

    nodes, A, C, E0, E1, E2 = _tc_encode(
        node_features, edge_features, W_node, b_node.reshape(1, H),
        Wmpa[0], Wmpc[0], Wmpb, W_edge, b_edge.reshape(1, H),
        b_mp.reshape(3, 1, H))

    for l, E in enumerate((E0, E1, E2)):
        nodes = _sc_layer(A, C, E, nodes, src, dst)
        if l < 2:
            A, C = _tc_proj(nodes, Wmpa[l + 1], Wmpc[l + 1])

    risk, ap = _tc_readout(
        nodes, W_r1, b_r1.reshape(1, H), W_r2, b_r2.reshape(1, 64),
        W_r3.reshape(1, 64), b_r3.reshape(1, 1), W_ap, b_ap.reshape(1, H))
    return (risk, ap)
